# baseline (speedup 1.0000x reference)
import functools

import jax
import jax.numpy as jnp
from jax import lax
from jax.experimental import pallas as pl
from jax.experimental.pallas import tpu as pltpu
from jax.experimental.pallas import tpu_sc as plsc

K = 8192
D = 256
N = 8192
RB = 256
NBLK = N // RB
CB = 1024
NCH = K // CB
TOPK = 8
BETA = 0.25


def _vq_main_body(zf_ref, w_ref, idx_ref, topk_ref, stats_ref, enc_ref,
                  d_scr, cnt_scr, acc_ref):
    pid = pl.program_id(0)

    z = zf_ref[...]
    z2 = jnp.sum(z * z, axis=1)
    zm2 = z * (-2.0)

    sum_d = jnp.float32(0.0)
    for c in range(NCH):
        wb = w_ref[c * CB:(c + 1) * CB, :]
        w2 = jnp.sum(wb * wb, axis=1)
        m = lax.dot_general(zm2, wb, (((1,), (1,)), ((), ())),
                            preferred_element_type=jnp.float32)
        d = (z2[:, None] + w2[None, :]) + m
        sum_d = sum_d + jnp.sum(d)
        d_scr[:, c * CB:(c + 1) * CB] = d

    iotaf = lax.broadcasted_iota(jnp.int32, (1, K), 1).astype(jnp.float32)
    idxs = []
    summin = None
    cnt = None
    for t in range(TOPK):
        dcur = d_scr[...]
        mv = jnp.min(dcur, axis=1, keepdims=True)
        iif = jnp.min(jnp.where(dcur == mv, iotaf, jnp.float32(K)),
                      axis=1, keepdims=True)
        idxs.append(iif)
        if t == 0:
            summin = jnp.sum(mv)
            oh = (iotaf == iif).astype(jnp.float32)
            enc_ref[...] = oh
            cnt = jnp.sum(oh, axis=0)
        if t < TOPK - 1:
            d_scr[...] = jnp.where(iotaf == iif, jnp.inf, dcur)

    idx_ref[...] = idxs[0].astype(jnp.int32)
    topk_ref[...] = jnp.concatenate(idxs, axis=1).astype(jnp.int32)

    @pl.when(pid == 0)
    def _init():
        cnt_scr[0, :] = jnp.zeros((K,), jnp.float32)
        acc_ref[0] = 0.0
        acc_ref[1] = 0.0

    cnt_scr[0, :] = cnt_scr[0, :] + cnt
    acc_ref[0] = acc_ref[0] + sum_d
    acc_ref[1] = acc_ref[1] + summin

    @pl.when(pid == NBLK - 1)
    def _fin():
        p = cnt_scr[0, :] / jnp.float32(N)
        ent = jnp.sum(p * jnp.log(p + 1e-10))
        perplexity = jnp.exp(-ent)
        mean_distance = acc_ref[0] / jnp.float32(N * K)
        loss = (1.0 + BETA) * acc_ref[1] / jnp.float32(N * D)
        lane = lax.broadcasted_iota(jnp.int32, (1, 128), 1)
        stats_ref[...] = jnp.where(
            lane == 0, loss,
            jnp.where(lane == 1, perplexity,
                      jnp.where(lane == 2, mean_distance, 0.0)))


def _onehot_body(idx_ref, enc_ref):
    iota = lax.broadcasted_iota(jnp.int32, (RB, K), 1)
    enc_ref[...] = (iota == idx_ref[...]).astype(jnp.float32)


@functools.lru_cache(maxsize=1)
def _make_gather():
    info = plsc.get_sparse_core_info()
    nc, ns = info.num_cores, info.num_subcores
    nw = nc * ns
    bpw = N // nw
    mesh = plsc.VectorSubcoreMesh(core_axis_name="c", subcore_axis_name="s")

    @functools.partial(
        pl.kernel, mesh=mesh,
        out_type=jax.ShapeDtypeStruct((N, D), jnp.float32),
        scratch_types=[
            pltpu.VMEM((bpw,), jnp.int32),
            pltpu.VMEM((bpw, D), jnp.float32),
            pltpu.SemaphoreType.DMA,
        ],
    )
    def gather(table_hbm, idx_hbm, out_hbm, idx_v, rows_v, sem):
        wid = lax.axis_index("s") * nc + lax.axis_index("c")
        base = wid * bpw
        pltpu.sync_copy(idx_hbm.at[pl.ds(base, bpw)], idx_v)
        pltpu.async_copy(table_hbm.at[idx_v], rows_v, sem).wait()
        pltpu.sync_copy(rows_v, out_hbm.at[pl.ds(base, bpw)])

    return gather


def kernel(z, W):
    zp = jnp.transpose(z, (0, 2, 3, 1))
    zf = zp.reshape(N, D)

    idx, topk, stats, min_encodings = pl.pallas_call(
        _vq_main_body,
        grid=(NBLK,),
        in_specs=[
            pl.BlockSpec((RB, D), lambda i: (i, 0)),
            pl.BlockSpec((K, D), lambda i: (0, 0)),
        ],
        out_specs=[
            pl.BlockSpec((RB, 1), lambda i: (i, 0)),
            pl.BlockSpec((RB, TOPK), lambda i: (i, 0)),
            pl.BlockSpec((1, 128), lambda i: (0, 0)),
            pl.BlockSpec((RB, K), lambda i: (i, 0)),
        ],
        out_shape=[
            jax.ShapeDtypeStruct((N, 1), jnp.int32),
            jax.ShapeDtypeStruct((N, TOPK), jnp.int32),
            jax.ShapeDtypeStruct((1, 128), jnp.float32),
            jax.ShapeDtypeStruct((N, K), jnp.float32),
        ],
        scratch_shapes=[
            pltpu.VMEM((RB, K), jnp.float32),
            pltpu.VMEM((1, K), jnp.float32),
            pltpu.SMEM((2,), jnp.float32),
        ],
    )(zf, W)

    z_q_flat = _make_gather()(W, idx.reshape(N))
    z_q = z_q_flat.reshape(8, 32, 32, D)
    z_q_out = jnp.transpose(z_q, (0, 3, 1, 2))

    loss = stats[0, 0]
    perplexity = stats[0, 1]
    mean_distance = stats[0, 2]

    return (z_q_out, loss, perplexity, min_encodings, idx, mean_distance,
            topk)


# speedup vs baseline: 1.0291x; 1.0291x over previous
import functools

import jax
import jax.numpy as jnp
from jax import lax
from jax.experimental import pallas as pl
from jax.experimental.pallas import tpu as pltpu
from jax.experimental.pallas import tpu_sc as plsc

K = 8192
D = 256
N = 8192
RB = 256
NBLK = N // RB
CB = 1024
NCH = K // CB
TOPK = 8
BETA = 0.25


def _vq_main_body(zf_ref, w_ref, idx_ref, topk_ref, stats_ref, enc_ref,
                  d_scr, cnt_scr, acc_ref):
    pid = pl.program_id(0)

    z = zf_ref[...]
    z2 = jnp.sum(z * z, axis=1)
    zm2 = z * (-2.0)

    for c in range(NCH):
        wb = w_ref[c * CB:(c + 1) * CB, :]
        w2 = jnp.sum(wb * wb, axis=1)
        m = lax.dot_general(zm2, wb, (((1,), (1,)), ((), ())),
                            preferred_element_type=jnp.float32)
        d_scr[:, c * CB:(c + 1) * CB] = (z2[:, None] + w2[None, :]) + m

    d0 = d_scr[...]
    sum_d = jnp.sum(d0)

    iotaf = lax.broadcasted_iota(jnp.int32, (1, K), 1).astype(jnp.float32)
    idxs = []
    summin = None
    cnt = None
    for t in range(TOPK):
        dcur = d_scr[...]
        mv = jnp.min(dcur, axis=1, keepdims=True)
        iif = jnp.min(jnp.where(dcur == mv, iotaf, jnp.float32(K)),
                      axis=1, keepdims=True)
        idxs.append(iif)
        if t == 0:
            summin = jnp.sum(mv)
            oh = (iotaf == iif).astype(jnp.float32)
            enc_ref[...] = oh
            cnt = jnp.sum(oh, axis=0)
        if t < TOPK - 1:
            d_scr[...] = jnp.where(iotaf == iif, jnp.inf, dcur)

    idx_ref[...] = idxs[0].astype(jnp.int32)
    topk_ref[...] = jnp.concatenate(idxs, axis=1).astype(jnp.int32)

    @pl.when(pid == 0)
    def _init():
        cnt_scr[0, :] = jnp.zeros((K,), jnp.float32)
        acc_ref[0] = 0.0
        acc_ref[1] = 0.0

    cnt_scr[0, :] = cnt_scr[0, :] + cnt
    acc_ref[0] = acc_ref[0] + sum_d
    acc_ref[1] = acc_ref[1] + summin

    @pl.when(pid == NBLK - 1)
    def _fin():
        p = cnt_scr[0, :] / jnp.float32(N)
        ent = jnp.sum(p * jnp.log(p + 1e-10))
        perplexity = jnp.exp(-ent)
        mean_distance = acc_ref[0] / jnp.float32(N * K)
        loss = (1.0 + BETA) * acc_ref[1] / jnp.float32(N * D)
        lane = lax.broadcasted_iota(jnp.int32, (1, 128), 1)
        stats_ref[...] = jnp.where(
            lane == 0, loss,
            jnp.where(lane == 1, perplexity,
                      jnp.where(lane == 2, mean_distance, 0.0)))


def _onehot_body(idx_ref, enc_ref):
    iota = lax.broadcasted_iota(jnp.int32, (RB, K), 1)
    enc_ref[...] = (iota == idx_ref[...]).astype(jnp.float32)


@functools.lru_cache(maxsize=1)
def _make_gather():
    info = plsc.get_sparse_core_info()
    nc, ns = info.num_cores, info.num_subcores
    nw = nc * ns
    bpw = N // nw
    mesh = plsc.VectorSubcoreMesh(core_axis_name="c", subcore_axis_name="s")

    @functools.partial(
        pl.kernel, mesh=mesh,
        out_type=jax.ShapeDtypeStruct((N, D), jnp.float32),
        scratch_types=[
            pltpu.VMEM((bpw,), jnp.int32),
            pltpu.VMEM((bpw, D), jnp.float32),
            pltpu.SemaphoreType.DMA,
        ],
    )
    def gather(table_hbm, idx_hbm, out_hbm, idx_v, rows_v, sem):
        wid = lax.axis_index("s") * nc + lax.axis_index("c")
        base = wid * bpw
        pltpu.sync_copy(idx_hbm.at[pl.ds(base, bpw)], idx_v)
        pltpu.async_copy(table_hbm.at[idx_v], rows_v, sem).wait()
        pltpu.sync_copy(rows_v, out_hbm.at[pl.ds(base, bpw)])

    return gather


def kernel(z, W):
    zp = jnp.transpose(z, (0, 2, 3, 1))
    zf = zp.reshape(N, D)

    idx, topk, stats, min_encodings = pl.pallas_call(
        _vq_main_body,
        grid=(NBLK,),
        in_specs=[
            pl.BlockSpec((RB, D), lambda i: (i, 0)),
            pl.BlockSpec((K, D), lambda i: (0, 0)),
        ],
        out_specs=[
            pl.BlockSpec((RB, 1), lambda i: (i, 0)),
            pl.BlockSpec((RB, TOPK), lambda i: (i, 0)),
            pl.BlockSpec((1, 128), lambda i: (0, 0)),
            pl.BlockSpec((RB, K), lambda i: (i, 0)),
        ],
        out_shape=[
            jax.ShapeDtypeStruct((N, 1), jnp.int32),
            jax.ShapeDtypeStruct((N, TOPK), jnp.int32),
            jax.ShapeDtypeStruct((1, 128), jnp.float32),
            jax.ShapeDtypeStruct((N, K), jnp.float32),
        ],
        scratch_shapes=[
            pltpu.VMEM((RB, K), jnp.float32),
            pltpu.VMEM((1, K), jnp.float32),
            pltpu.SMEM((2,), jnp.float32),
        ],
    )(zf, W)

    z_q_flat = _make_gather()(W, idx.reshape(N))
    z_q = z_q_flat.reshape(8, 32, 32, D)
    z_q_out = jnp.transpose(z_q, (0, 3, 1, 2))

    loss = stats[0, 0]
    perplexity = stats[0, 1]
    mean_distance = stats[0, 2]

    return (z_q_out, loss, perplexity, min_encodings, idx, mean_distance,
            topk)
